# baseline (speedup 1.0000x reference)
import jax
import jax.numpy as jnp
from jax import lax
from jax.experimental import pallas as pl
from jax.experimental.pallas import tpu as pltpu
from jax.experimental.pallas import tpu_sc as plsc

B = 16384
D = 64
NW = 32
BPW = B // NW
L = 16
NU = 1000000


def _body(uids, iids, uembT, iembT, ubias, ibias, out,
          uidx_v, iidx_v, ue_v, ie_v, ub_v, ib_v, out_v, sem):
    wid = lax.axis_index("s") * 2 + lax.axis_index("c")
    base = wid * BPW

    pltpu.sync_copy(uids.at[pl.ds(base, BPW)], uidx_v)
    pltpu.sync_copy(iids.at[pl.ds(base, BPW)], iidx_v)

    copies = []
    for d in range(D):
        copies.append(
            pltpu.async_copy(uembT.at[d].at[uidx_v], ue_v.at[d], sem))
        copies.append(
            pltpu.async_copy(iembT.at[d].at[iidx_v], ie_v.at[d], sem))
    copies.append(pltpu.async_copy(ubias.at[uidx_v], ub_v, sem))
    copies.append(pltpu.async_copy(ibias.at[iidx_v], ib_v, sem))
    for c in copies:
        c.wait()

    def group(g, carry):
        s = pl.ds(g * L, L)
        acc = ub_v[s] + ib_v[s]
        for d in range(D):
            acc = acc + ue_v[d, s] * ie_v[d, s]
        out_v[s] = acc
        return carry

    lax.fori_loop(0, BPW // L, group, 0)
    pltpu.sync_copy(out_v, out.at[pl.ds(base, BPW)])


def kernel(user_ids, item_ids, user_embed, item_embed, user_bias, item_bias):
    uids = user_ids.astype(jnp.int32)
    iids = item_ids.astype(jnp.int32)
    ueT = user_embed.T
    ieT = item_embed.T
    ub1 = user_bias.reshape(-1)
    ib1 = item_bias.reshape(-1)

    mesh = plsc.VectorSubcoreMesh(core_axis_name="c", subcore_axis_name="s")
    f = pl.kernel(
        _body,
        mesh=mesh,
        out_type=jax.ShapeDtypeStruct((B,), jnp.float32),
        compiler_params=pltpu.CompilerParams(
            needs_layout_passes=False, use_tc_tiling_on_sc=False
        ),
        scratch_types=[
            pltpu.VMEM((BPW,), jnp.int32),
            pltpu.VMEM((BPW,), jnp.int32),
            pltpu.VMEM((D, BPW), jnp.float32),
            pltpu.VMEM((D, BPW), jnp.float32),
            pltpu.VMEM((BPW,), jnp.float32),
            pltpu.VMEM((BPW,), jnp.float32),
            pltpu.VMEM((BPW,), jnp.float32),
            pltpu.SemaphoreType.DMA,
        ],
    )
    return f(uids, iids, ueT, ieT, ub1, ib1)


# speedup vs baseline: 21.2479x; 21.2479x over previous
import jax
import jax.numpy as jnp
from jax import lax
from jax.experimental import pallas as pl
from jax.experimental.pallas import tpu as pltpu
from jax.experimental.pallas import tpu_sc as plsc

B = 16384
D = 64
NW = 32
BPW = B // NW
L = 16
SG = 4


def _dot_body(uids, iids, uembT, iembT, out,
              uidx_v, iidx_v, slab_u, slab_i, out_v, sem):
    wid = lax.axis_index("s") * 2 + lax.axis_index("c")
    base = wid * BPW

    pltpu.sync_copy(uids.at[pl.ds(base, BPW)], uidx_v)
    pltpu.sync_copy(iids.at[pl.ds(base, BPW)], iidx_v)

    dvec = lax.iota(jnp.int32, L)
    lane = lax.iota(jnp.int32, L)

    def group(g, carry):
        idu = uidx_v[pl.ds(g * L, L)]
        idi = iidx_v[pl.ds(g * L, L)]
        accv = jnp.zeros((L,), jnp.float32)
        for sub in range(L // SG):
            copies = []
            offs = []
            for j in range(SG):
                uid = idu[sub * SG + j]
                iid = idi[sub * SG + j]
                cu = pl.multiple_of((uid >> 7) << 7, 128)
                ci = pl.multiple_of((iid >> 7) << 7, 128)
                copies.append(pltpu.async_copy(
                    uembT.at[:, pl.ds(cu, 128)], slab_u.at[j], sem))
                copies.append(pltpu.async_copy(
                    iembT.at[:, pl.ds(ci, 128)], slab_i.at[j], sem))
                offs.append((uid & 127, iid & 127))
            for c in copies:
                c.wait()
            for j in range(SG):
                ou = jnp.full((L,), offs[j][0], jnp.int32)
                oi = jnp.full((L,), offs[j][1], jnp.int32)
                acc = jnp.zeros((L,), jnp.float32)
                for k in range(D // L):
                    u = plsc.load_gather(slab_u.at[j], [dvec + k * L, ou])
                    i = plsc.load_gather(slab_i.at[j], [dvec + k * L, oi])
                    acc = acc + u * i
                s = jnp.sum(acc)
                accv = jnp.where(lane == sub * SG + j, s, accv)
        out_v[pl.ds(g * L, L)] = accv
        return carry

    lax.fori_loop(0, BPW // L, group, 0)
    pltpu.sync_copy(out_v, out.at[pl.ds(base, BPW)])


def _bias_body(uids, iids, ubias, ibias, partial, out,
               uidx_v, iidx_v, ub_v, ib_v, p_v, sem):
    wid = lax.axis_index("s") * 2 + lax.axis_index("c")
    base = wid * BPW

    pltpu.sync_copy(uids.at[pl.ds(base, BPW)], uidx_v)
    pltpu.sync_copy(iids.at[pl.ds(base, BPW)], iidx_v)
    pltpu.sync_copy(partial.at[pl.ds(base, BPW)], p_v)
    c1 = pltpu.async_copy(ubias.at[uidx_v], ub_v, sem)
    c2 = pltpu.async_copy(ibias.at[iidx_v], ib_v, sem)
    c1.wait()
    c2.wait()

    def group(g, carry):
        s = pl.ds(g * L, L)
        p_v[s] = p_v[s] + ub_v[s] + ib_v[s]
        return carry

    lax.fori_loop(0, BPW // L, group, 0)
    pltpu.sync_copy(p_v, out.at[pl.ds(base, BPW)])


def kernel(user_ids, item_ids, user_embed, item_embed, user_bias, item_bias):
    uids = user_ids.astype(jnp.int32)
    iids = item_ids.astype(jnp.int32)
    ueT = user_embed.T
    ieT = item_embed.T
    ub1 = user_bias.reshape(-1)
    ib1 = item_bias.reshape(-1)

    mesh = plsc.VectorSubcoreMesh(core_axis_name="c", subcore_axis_name="s")
    dot = pl.kernel(
        _dot_body,
        mesh=mesh,
        out_type=jax.ShapeDtypeStruct((B,), jnp.float32),
        compiler_params=pltpu.CompilerParams(
            needs_layout_passes=False, use_tc_tiling_on_sc=True
        ),
        scratch_types=[
            pltpu.VMEM((BPW,), jnp.int32),
            pltpu.VMEM((BPW,), jnp.int32),
            pltpu.VMEM((SG, D, 128), jnp.float32),
            pltpu.VMEM((SG, D, 128), jnp.float32),
            pltpu.VMEM((BPW,), jnp.float32),
            pltpu.SemaphoreType.DMA,
        ],
    )
    partial = dot(uids, iids, ueT, ieT)

    biased = pl.kernel(
        _bias_body,
        mesh=mesh,
        out_type=jax.ShapeDtypeStruct((B,), jnp.float32),
        compiler_params=pltpu.CompilerParams(
            needs_layout_passes=False, use_tc_tiling_on_sc=False
        ),
        scratch_types=[
            pltpu.VMEM((BPW,), jnp.int32),
            pltpu.VMEM((BPW,), jnp.int32),
            pltpu.VMEM((BPW,), jnp.float32),
            pltpu.VMEM((BPW,), jnp.float32),
            pltpu.VMEM((BPW,), jnp.float32),
            pltpu.SemaphoreType.DMA,
        ],
    )
    return biased(uids, iids, ub1, ib1, partial)


# speedup vs baseline: 25.5627x; 1.2031x over previous
import jax
import jax.numpy as jnp
from jax import lax
from jax.experimental import pallas as pl
from jax.experimental.pallas import tpu as pltpu
from jax.experimental.pallas import tpu_sc as plsc

B = 16384
D = 64
NW = 32
BPW = B // NW
L = 16
SG = 4


def _dot_body(uids, iids, uembT, iembT, out,
              uidx_v, iidx_v, slabs, ucol_v, icol_v, out_v, sem):
    wid = lax.axis_index("s") * 2 + lax.axis_index("c")
    base = wid * BPW

    pltpu.sync_copy(uids.at[pl.ds(base, BPW)], uidx_v)
    pltpu.sync_copy(iids.at[pl.ds(base, BPW)], iidx_v)

    dvec = lax.iota(jnp.int32, L)
    lane = lax.iota(jnp.int32, L)
    NSUB = L // SG

    def group(g, carry):
        idu = uidx_v[pl.ds(g * L, L)]
        idi = iidx_v[pl.ds(g * L, L)]

        def issue(p):
            q, ids = divmod(p, 2)
            idvec = idi if ids else idu
            table = iembT if ids else uembT
            cps, offs = [], []
            for j in range(SG):
                tid = idvec[q * SG + j]
                c0 = pl.multiple_of((tid >> 7) << 7, 128)
                cps.append(pltpu.async_copy(
                    table.at[:, pl.ds(c0, 128)], slabs.at[p % 2, j], sem))
                offs.append(tid & 127)
            return cps, offs

        def extract(p, offs, col_v):
            for j in range(SG):
                o = jnp.full((L,), offs[j], jnp.int32)
                for k in range(D // L):
                    col_v[pl.ds(j * D + k * L, L)] = plsc.load_gather(
                        slabs.at[p % 2, j], [dvec + k * L, o])

        accv = jnp.zeros((L,), jnp.float32)
        pend = issue(0)
        for p in range(2 * NSUB):
            nxt = issue(p + 1) if p + 1 < 2 * NSUB else None
            cps, offs = pend
            for c in cps:
                c.wait()
            q, ids = divmod(p, 2)
            extract(p, offs, icol_v if ids else ucol_v)
            if ids:
                for j in range(SG):
                    acc = (ucol_v[pl.ds(j * D, L)] * icol_v[pl.ds(j * D, L)])
                    for k in range(1, D // L):
                        acc = acc + (ucol_v[pl.ds(j * D + k * L, L)]
                                     * icol_v[pl.ds(j * D + k * L, L)])
                    s = jnp.sum(acc)
                    accv = jnp.where(lane == q * SG + j, s, accv)
            pend = nxt
        out_v[pl.ds(g * L, L)] = accv
        return carry

    lax.fori_loop(0, BPW // L, group, 0)
    pltpu.sync_copy(out_v, out.at[pl.ds(base, BPW)])


def _bias_body(uids, iids, ubias, ibias, partial, out,
               uidx_v, iidx_v, ub_v, ib_v, p_v, sem):
    wid = lax.axis_index("s") * 2 + lax.axis_index("c")
    base = wid * BPW

    pltpu.sync_copy(uids.at[pl.ds(base, BPW)], uidx_v)
    pltpu.sync_copy(iids.at[pl.ds(base, BPW)], iidx_v)
    pltpu.sync_copy(partial.at[pl.ds(base, BPW)], p_v)
    c1 = pltpu.async_copy(ubias.at[uidx_v], ub_v, sem)
    c2 = pltpu.async_copy(ibias.at[iidx_v], ib_v, sem)
    c1.wait()
    c2.wait()

    def group(g, carry):
        s = pl.ds(g * L, L)
        p_v[s] = p_v[s] + ub_v[s] + ib_v[s]
        return carry

    lax.fori_loop(0, BPW // L, group, 0)
    pltpu.sync_copy(p_v, out.at[pl.ds(base, BPW)])


def kernel(user_ids, item_ids, user_embed, item_embed, user_bias, item_bias):
    uids = user_ids.astype(jnp.int32)
    iids = item_ids.astype(jnp.int32)
    ueT = user_embed.T
    ieT = item_embed.T
    ub1 = user_bias.reshape(-1)
    ib1 = item_bias.reshape(-1)

    mesh = plsc.VectorSubcoreMesh(core_axis_name="c", subcore_axis_name="s")
    dot = pl.kernel(
        _dot_body,
        mesh=mesh,
        out_type=jax.ShapeDtypeStruct((B,), jnp.float32),
        compiler_params=pltpu.CompilerParams(
            needs_layout_passes=False, use_tc_tiling_on_sc=True
        ),
        scratch_types=[
            pltpu.VMEM((BPW,), jnp.int32),
            pltpu.VMEM((BPW,), jnp.int32),
            pltpu.VMEM((2, SG, D, 128), jnp.float32),
            pltpu.VMEM((SG * D,), jnp.float32),
            pltpu.VMEM((SG * D,), jnp.float32),
            pltpu.VMEM((BPW,), jnp.float32),
            pltpu.SemaphoreType.DMA,
        ],
    )
    partial = dot(uids, iids, ueT, ieT)

    biased = pl.kernel(
        _bias_body,
        mesh=mesh,
        out_type=jax.ShapeDtypeStruct((B,), jnp.float32),
        compiler_params=pltpu.CompilerParams(
            needs_layout_passes=False, use_tc_tiling_on_sc=False
        ),
        scratch_types=[
            pltpu.VMEM((BPW,), jnp.int32),
            pltpu.VMEM((BPW,), jnp.int32),
            pltpu.VMEM((BPW,), jnp.float32),
            pltpu.VMEM((BPW,), jnp.float32),
            pltpu.VMEM((BPW,), jnp.float32),
            pltpu.SemaphoreType.DMA,
        ],
    )
    return biased(uids, iids, ub1, ib1, partial)


# speedup vs baseline: 26.1043x; 1.0212x over previous
import jax
import jax.numpy as jnp
from jax import lax
from jax.experimental import pallas as pl
from jax.experimental.pallas import tpu as pltpu
from jax.experimental.pallas import tpu_sc as plsc

B = 16384
D = 64
NW = 32
BPW = B // NW
L = 16
SG = 4


def _dot_body(uids, iids, uembT, iembT, out,
              uidx_v, iidx_v, slabs, ucol_v, icol_v, out_v, sem):
    wid = lax.axis_index("s") * 2 + lax.axis_index("c")
    base = wid * BPW

    pltpu.sync_copy(uids.at[pl.ds(base, BPW)], uidx_v)
    pltpu.sync_copy(iids.at[pl.ds(base, BPW)], iidx_v)

    dvec = lax.iota(jnp.int32, L)
    lane = lax.iota(jnp.int32, L)
    GIDS = 64
    NPH = 2 * (GIDS // SG)
    NBUF = 3

    def group(g, carry):
        idus = [uidx_v[pl.ds(g * GIDS + v * L, L)] for v in range(GIDS // L)]
        idis = [iidx_v[pl.ds(g * GIDS + v * L, L)] for v in range(GIDS // L)]

        def issue(p):
            q, ids = divmod(p, 2)
            idvecs = idis if ids else idus
            table = iembT if ids else uembT
            cps, offs = [], []
            for j in range(SG):
                e = q * SG + j
                tid = idvecs[e // L][e % L]
                c0 = pl.multiple_of((tid >> 7) << 7, 128)
                cps.append(pltpu.async_copy(
                    table.at[:, pl.ds(c0, 128)], slabs.at[p % NBUF, j], sem))
                offs.append(tid & 127)
            return cps, offs

        def extract(p, offs, col_v):
            for j in range(SG):
                o = jnp.full((L,), offs[j], jnp.int32)
                for k in range(D // L):
                    col_v[pl.ds(j * D + k * L, L)] = plsc.load_gather(
                        slabs.at[p % NBUF, j], [dvec + k * L, o])

        accvs = [jnp.zeros((L,), jnp.float32) for _ in range(GIDS // L)]
        ring = [issue(0), issue(1)]
        for p in range(NPH):
            if p + 2 < NPH:
                ring.append(issue(p + 2))
            cps, offs = ring[p]
            for c in cps:
                c.wait()
            q, ids = divmod(p, 2)
            extract(p, offs, icol_v if ids else ucol_v)
            if ids:
                for j in range(SG):
                    acc = (ucol_v[pl.ds(j * D, L)] * icol_v[pl.ds(j * D, L)])
                    for k in range(1, D // L):
                        acc = acc + (ucol_v[pl.ds(j * D + k * L, L)]
                                     * icol_v[pl.ds(j * D + k * L, L)])
                    s = jnp.sum(acc)
                    e = q * SG + j
                    accvs[e // L] = jnp.where(
                        lane == e % L, s, accvs[e // L])
        for v in range(GIDS // L):
            out_v[pl.ds(g * GIDS + v * L, L)] = accvs[v]
        return carry

    lax.fori_loop(0, BPW // GIDS, group, 0)
    pltpu.sync_copy(out_v, out.at[pl.ds(base, BPW)])


def _bias_body(uids, iids, ubias, ibias, partial, out,
               uidx_v, iidx_v, ub_v, ib_v, p_v, sem):
    wid = lax.axis_index("s") * 2 + lax.axis_index("c")
    base = wid * BPW

    pltpu.sync_copy(uids.at[pl.ds(base, BPW)], uidx_v)
    pltpu.sync_copy(iids.at[pl.ds(base, BPW)], iidx_v)
    pltpu.sync_copy(partial.at[pl.ds(base, BPW)], p_v)
    c1 = pltpu.async_copy(ubias.at[uidx_v], ub_v, sem)
    c2 = pltpu.async_copy(ibias.at[iidx_v], ib_v, sem)
    c1.wait()
    c2.wait()

    def group(g, carry):
        s = pl.ds(g * L, L)
        p_v[s] = p_v[s] + ub_v[s] + ib_v[s]
        return carry

    lax.fori_loop(0, BPW // L, group, 0)
    pltpu.sync_copy(p_v, out.at[pl.ds(base, BPW)])


def kernel(user_ids, item_ids, user_embed, item_embed, user_bias, item_bias):
    uids = user_ids.astype(jnp.int32)
    iids = item_ids.astype(jnp.int32)
    ueT = user_embed.T
    ieT = item_embed.T
    ub1 = user_bias.reshape(-1)
    ib1 = item_bias.reshape(-1)

    mesh = plsc.VectorSubcoreMesh(core_axis_name="c", subcore_axis_name="s")
    dot = pl.kernel(
        _dot_body,
        mesh=mesh,
        out_type=jax.ShapeDtypeStruct((B,), jnp.float32),
        compiler_params=pltpu.CompilerParams(
            needs_layout_passes=False, use_tc_tiling_on_sc=True
        ),
        scratch_types=[
            pltpu.VMEM((BPW,), jnp.int32),
            pltpu.VMEM((BPW,), jnp.int32),
            pltpu.VMEM((3, SG, D, 128), jnp.float32),
            pltpu.VMEM((SG * D,), jnp.float32),
            pltpu.VMEM((SG * D,), jnp.float32),
            pltpu.VMEM((BPW,), jnp.float32),
            pltpu.SemaphoreType.DMA,
        ],
    )
    partial = dot(uids, iids, ueT, ieT)

    biased = pl.kernel(
        _bias_body,
        mesh=mesh,
        out_type=jax.ShapeDtypeStruct((B,), jnp.float32),
        compiler_params=pltpu.CompilerParams(
            needs_layout_passes=False, use_tc_tiling_on_sc=False
        ),
        scratch_types=[
            pltpu.VMEM((BPW,), jnp.int32),
            pltpu.VMEM((BPW,), jnp.int32),
            pltpu.VMEM((BPW,), jnp.float32),
            pltpu.VMEM((BPW,), jnp.float32),
            pltpu.VMEM((BPW,), jnp.float32),
            pltpu.SemaphoreType.DMA,
        ],
    )
    return biased(uids, iids, ub1, ib1, partial)
